# Initial kernel scaffold; baseline (speedup 1.0000x reference)
import jax
import jax.numpy as jnp
from jax.experimental import pallas as pl


def kernel(u_pos, v_pos, v_neg, batch_size, u_weight, v_weight):
    raise NotImplementedError("write your pallas kernel here")


# speedup vs baseline: 2.9718x; 2.9718x over previous
import functools

import jax
import jax.numpy as jnp
from jax import lax
from jax.experimental import pallas as pl
from jax.experimental.pallas import tpu as pltpu
from jax.experimental.pallas import tpu_sc as plsc

DIM = 64
NEG = 10
NC = 2
NS = 16
NW = NC * NS
LANES = 16
CHUNK = 128


def _sc_scores(u_weight, v_weight, u_pos, v_pos, v_neg_flat, batch):
    bpw = batch // NW
    nchunks = bpw // CHUNK
    mesh = plsc.VectorSubcoreMesh(
        core_axis_name="c", subcore_axis_name="s", num_cores=NC, num_subcores=NS
    )

    @functools.partial(
        pl.kernel,
        out_type=[
            jax.ShapeDtypeStruct((batch,), jnp.float32),
            jax.ShapeDtypeStruct((batch,), jnp.float32),
        ],
        mesh=mesh,
        compiler_params=pltpu.CompilerParams(
            needs_layout_passes=False, use_tc_tiling_on_sc=False),
        scratch_types=[
            pltpu.VMEM((CHUNK,), jnp.int32),
            pltpu.VMEM((CHUNK,), jnp.int32),
            pltpu.VMEM((NEG * CHUNK,), jnp.int32),
            pltpu.VMEM((CHUNK, DIM), jnp.float32),
            pltpu.VMEM((CHUNK, DIM), jnp.float32),
            pltpu.VMEM((NEG * CHUNK, DIM), jnp.float32),
            pltpu.VMEM((CHUNK * LANES,), jnp.float32),
            pltpu.VMEM((CHUNK * LANES,), jnp.float32),
            pltpu.VMEM((CHUNK,), jnp.float32),
            pltpu.VMEM((CHUNK,), jnp.float32),
            pltpu.SemaphoreType.DMA,
        ],
    )
    def sc_kernel(u_w, v_w, up, vp, vn, score_out, nscore_out,
                  idx_u, idx_v, idx_n, rows_u, rows_v, rows_n,
                  sc_part, nc_part, sc_chunk, nc_chunk, sem):
        wid = lax.axis_index("s") * NC + lax.axis_index("c")
        base = wid * bpw
        for c in range(nchunks):
            off = base + c * CHUNK
            pltpu.sync_copy(up.at[pl.ds(off, CHUNK)], idx_u)
            pltpu.sync_copy(vp.at[pl.ds(off, CHUNK)], idx_v)
            pltpu.sync_copy(vn.at[pl.ds(off * NEG, CHUNK * NEG)], idx_n)
            cps = [
                pltpu.async_copy(u_w.at[idx_u], rows_u, sem),
                pltpu.async_copy(v_w.at[idx_v], rows_v, sem),
            ]
            for j in range(NEG):
                cps.append(
                    pltpu.async_copy(
                        v_w.at[idx_n.at[pl.ds(j * CHUNK, CHUNK)]],
                        rows_n.at[pl.ds(j * CHUNK, CHUNK)],
                        sem,
                    )
                )
            for cp in cps:
                cp.wait()

            def body(b, _):
                acc = None
                nacc = None
                for k in range(DIM // LANES):
                    sl = pl.ds(k * LANES, LANES)
                    u = rows_u[b, sl]
                    v = rows_v[b, sl]
                    nsum = rows_n[b * NEG, sl]
                    for n in range(1, NEG):
                        nsum = nsum + rows_n[b * NEG + n, sl]
                    pk = u * v
                    npk = u * nsum
                    acc = pk if acc is None else acc + pk
                    nacc = npk if nacc is None else nacc + npk
                sc_part[pl.ds(b * LANES, LANES)] = acc
                nc_part[pl.ds(b * LANES, LANES)] = nacc
                return 0

            lax.fori_loop(0, CHUNK, body, 0)

            lane_iota = lax.iota(jnp.int32, LANES)
            for g in range(CHUNK // LANES):
                rowi = (g * LANES + lane_iota) * LANES
                acc_s = plsc.load_gather(sc_part, [rowi])
                acc_n = plsc.load_gather(nc_part, [rowi])
                for k in range(1, LANES):
                    acc_s = acc_s + plsc.load_gather(sc_part, [rowi + k])
                    acc_n = acc_n + plsc.load_gather(nc_part, [rowi + k])
                sc_chunk[pl.ds(g * LANES, LANES)] = acc_s
                nc_chunk[pl.ds(g * LANES, LANES)] = acc_n
            pltpu.sync_copy(sc_chunk, score_out.at[pl.ds(off, CHUNK)])
            pltpu.sync_copy(nc_chunk, nscore_out.at[pl.ds(off, CHUNK)])

    return sc_kernel(u_weight, v_weight, u_pos, v_pos, v_neg_flat)


def _tc_loss_body(s_ref, n_ref, o_ref):
    s = s_ref[...]
    n = n_ref[...]
    val = jax.nn.log_sigmoid(s) + jax.nn.log_sigmoid(-n)
    o_ref[0, 0] = -jnp.sum(val)


def kernel(u_pos, v_pos, v_neg, batch_size, u_weight, v_weight):
    batch = u_pos.shape[0]
    scores, nscores = _sc_scores(
        u_weight,
        v_weight,
        u_pos.astype(jnp.int32),
        v_pos.astype(jnp.int32),
        v_neg.reshape(-1).astype(jnp.int32),
        batch,
    )
    rows = batch // 128
    loss_sum = pl.pallas_call(
        _tc_loss_body,
        out_shape=jax.ShapeDtypeStruct((1, 1), jnp.float32),
        out_specs=pl.BlockSpec(memory_space=pltpu.SMEM),
    )(scores.reshape(rows, 128), nscores.reshape(rows, 128))
    return loss_sum[0, 0] / batch_size
